# baseline (speedup 1.0000x reference)
import functools

import jax
import jax.numpy as jnp
from jax import lax
from jax.experimental import pallas as pl
from jax.experimental.pallas import tpu as pltpu
from jax.experimental.pallas import tpu_sc as plsc

_B = 1024
_T = 200
_D = 64
_CHUNK = 40
_SEQ_CHUNKS = _T // _CHUNK
_NCHUNKS = _B * _SEQ_CHUNKS


def _make_sc_kernel():
    info = plsc.get_sparse_core_info()
    nc, ns = info.num_cores, info.num_subcores
    nw = nc * ns
    cpw = _NCHUNKS // nw
    rpw = _B // nw

    mesh = plsc.VectorSubcoreMesh(core_axis_name="c", subcore_axis_name="s")

    @functools.partial(
        pl.kernel,
        mesh=mesh,
        out_type=jax.ShapeDtypeStruct((_B * _T, _D), jnp.float32),
        scratch_types=[
            pltpu.VMEM((rpw * _T,), jnp.int32),
            pltpu.VMEM((_T, _D), jnp.float32),
            pltpu.VMEM((_CHUNK, 2 * _D), jnp.float32),
            pltpu.VMEM((_CHUNK, 2 * _D), jnp.float32),
            pltpu.SemaphoreType.DMA,
            pltpu.SemaphoreType.DMA,
            pltpu.SemaphoreType.DMA,
            pltpu.SemaphoreType.DMA,
        ],
        compiler_params=pltpu.CompilerParams(use_tc_tiling_on_sc=False),
    )
    def k(idx_hbm, word_hbm, pos_hbm, out_hbm,
          idx_v, pos_v, b0, b1, gs0, gs1, ws0, ws1):
        bufs, gs, ws = (b0, b1), (gs0, gs1), (ws0, ws1)
        wid = lax.axis_index("s") * nc + lax.axis_index("c")
        row0 = wid * rpw
        pltpu.sync_copy(idx_hbm.at[pl.ds(8 * (row0 * (_T // 8)), rpw * _T)], idx_v)
        pltpu.sync_copy(pos_hbm, pos_v)

        def gather(c, h, b):
            r = lax.div(c, _SEQ_CHUNKS)
            pltpu.async_copy(
                word_hbm.at[idx_v.at[pl.ds(8 * (r * (_T // 8) + h * (_CHUNK // 8)), _CHUNK)]],
                bufs[b], gs[b])

        def wait_gather(b):
            pltpu.make_async_copy(
                bufs[b], word_hbm.at[pl.ds(0, _CHUNK)], gs[b]).wait()

        def write(c, h, b):
            r = lax.div(c, _SEQ_CHUNKS)
            pltpu.async_copy(
                bufs[b].at[:, pl.ds(0, _D)],
                out_hbm.at[pl.ds(8 * ((row0 + r) * (_T // 8) + h * (_CHUNK // 8)), _CHUNK)],
                ws[b])

        def wait_write(b):
            pltpu.make_async_copy(
                bufs[b].at[:, pl.ds(0, _D)], out_hbm.at[pl.ds(0, _CHUNK)],
                ws[b]).wait()

        def add_pos(h, b):
            buf = bufs[b]
            t0 = h * _CHUNK

            def row(r, _):
                for d in range(_D // 16):
                    sl = pl.ds(d * 16, 16)
                    plsc.addupdate(buf.at[r, sl], pos_v[t0 + r, sl])
                return ()

            lax.fori_loop(0, _CHUNK, row, (), unroll=4)

        def steady(c, h, h_next, b):
            wait_gather(b)
            add_pos(h, b)
            wait_write(1 - b)
            gather(c + 1, h_next, 1 - b)
            write(c, h, b)

        gather(0, 0, 0)
        wait_gather(0)
        add_pos(0, 0)
        gather(1, 1, 1)
        write(0, 0, 0)

        @pl.loop(1, 151, step=10)
        def _ten(c0):
            for j in range(10):
                steady(c0 + j, (1 + j) % 5, (2 + j) % 5, (1 + j) % 2)

        for c in range(151, cpw - 1):
            steady(c, c % 5, (c + 1) % 5, c % 2)

        wait_gather(1)
        add_pos(4, 1)
        write(cpw - 1, 4, 1)
        wait_write(0)
        wait_write(1)

    return k


_sc_kernel = _make_sc_kernel()


@jax.jit
def kernel(x, word_table, pos_table):
    word128 = jnp.pad(word_table, ((0, 0), (0, _D)))
    out = _sc_kernel(
        x.reshape(_B * _T).astype(jnp.int32), word128, pos_table)
    return out.reshape(_B, _T, _D)


# speedup vs baseline: 1.0951x; 1.0951x over previous
import functools

import jax
import jax.numpy as jnp
from jax import lax
from jax.experimental import pallas as pl
from jax.experimental.pallas import tpu as pltpu
from jax.experimental.pallas import tpu_sc as plsc

_B = 1024
_T = 200
_D = 64
_CHUNK = 100
_NCHUNKS = _B * _T // _CHUNK
_SEQ_CHUNKS = _T // _CHUNK


def _make_sc_kernel():
    info = plsc.get_sparse_core_info()
    nc, ns = info.num_cores, info.num_subcores
    nw = nc * ns
    cpw = _NCHUNKS // nw

    mesh = plsc.VectorSubcoreMesh(core_axis_name="c", subcore_axis_name="s")

    @functools.partial(
        pl.kernel,
        mesh=mesh,
        out_type=jax.ShapeDtypeStruct((_NCHUNKS, _CHUNK, _D), jnp.float32),
        scratch_types=[
            pltpu.VMEM((cpw, _CHUNK), jnp.int32),
            pltpu.VMEM((_T, _D), jnp.float32),
            pltpu.VMEM((_CHUNK, _D), jnp.float32),
            pltpu.VMEM((_CHUNK, _D), jnp.float32),
            pltpu.SemaphoreType.DMA,
            pltpu.SemaphoreType.DMA,
            pltpu.SemaphoreType.DMA,
            pltpu.SemaphoreType.DMA,
        ],
        compiler_params=pltpu.CompilerParams(use_tc_tiling_on_sc=False),
    )
    def k(idx_hbm, word_hbm, pos_hbm, out_hbm,
          idx_v, pos_v, b0, b1, gs0, gs1, ws0, ws1):
        bufs, gs, ws = (b0, b1), (gs0, gs1), (ws0, ws1)
        wid = lax.axis_index("s") * nc + lax.axis_index("c")
        base = wid * cpw
        pltpu.sync_copy(idx_hbm.at[pl.ds(base, cpw)], idx_v)
        pltpu.sync_copy(pos_hbm, pos_v)

        def gather(c, b):
            pltpu.async_copy(word_hbm.at[idx_v.at[c]], bufs[b], gs[b])

        def wait_gather(b):
            pltpu.make_async_copy(bufs[b], out_hbm.at[0], gs[b]).wait()

        def write(c, b):
            pltpu.async_copy(bufs[b], out_hbm.at[base + c], ws[b])

        def wait_write(b):
            pltpu.make_async_copy(bufs[b], out_hbm.at[0], ws[b]).wait()

        def add_pos(c, b):
            buf = bufs[b]
            t0 = lax.rem(base + c, _SEQ_CHUNKS) * _CHUNK

            def row(r, _):
                for d in range(_D // 16):
                    sl = pl.ds(d * 16, 16)
                    plsc.addupdate(buf.at[r, sl], pos_v[t0 + r, sl])
                return ()

            lax.fori_loop(0, _CHUNK, row, (), unroll=4)

        gather(0, 0)
        wait_gather(0)
        add_pos(0, 0)
        gather(1, 1)
        write(0, 0)

        @pl.loop(1, cpw - 1, step=2)
        def _pair(c0):
            for j in range(2):
                c = c0 + j
                b = (1 + j) % 2
                wait_gather(b)
                add_pos(c, b)
                wait_write(1 - b)
                gather(c + 1, 1 - b)
                write(c, b)

        wait_gather(1)
        add_pos(cpw - 1, 1)
        write(cpw - 1, 1)
        wait_write(0)
        wait_write(1)

    return k


_sc_kernel = _make_sc_kernel()


@jax.jit
def kernel(x, word_table, pos_table):
    idx = x.reshape(_NCHUNKS, _CHUNK).astype(jnp.int32)
    out = _sc_kernel(idx, word_table, pos_table)
    return out.reshape(_B, _T, _D)
